# Initial kernel scaffold; baseline (speedup 1.0000x reference)
import jax
import jax.numpy as jnp
from jax.experimental import pallas as pl


def kernel(desc0, desc1, W, b, alpha):
    raise NotImplementedError("write your pallas kernel here")


# speedup vs baseline: 14.7934x; 14.7934x over previous
import functools

import jax
import jax.numpy as jnp
from jax.experimental import pallas as pl

M = 4096
N = 4096
DIM = 256
ITERS = 5
THRESHOLD = 0.05
MN = float(M + N)
INV_MN = 1.0 / MN
MU_BIN = N / MN
BM = 256
PW = 512
NPANEL = N // PW
HIGH = jax.lax.Precision.HIGHEST


def _proj_body(d0_ref, d1_ref, w_ref, b_ref, md0_ref, md1_ref):
    w = w_ref[...]
    bias = b_ref[...]
    md0 = jax.lax.dot_general(d0_ref[...], w, (((1,), (1,)), ((), ())),
                              preferred_element_type=jnp.float32)
    md1 = jax.lax.dot_general(d1_ref[...], w, (((1,), (1,)), ((), ())),
                              preferred_element_type=jnp.float32)
    md0_ref[...] = (md0 + bias) * 0.25
    md1_ref[...] = (md1 + bias) * 0.25


def _kn_body(md0_ref, md1_ref, kn_ref, e_ref, s1_ref):
    k = jax.lax.dot_general(md0_ref[...], md1_ref[...], (((1,), (1,)), ((), ())),
                            preferred_element_type=jnp.float32)
    kn_ref[...] = k
    e = jnp.exp(k)
    e_ref[...] = e
    s1_ref[0, 0, :] = jnp.sum(e, axis=1)


def _sweep_body(e_ref, arow_ref, scal_ref, bout_ref, racc_ref):
    t = pl.program_id(0)
    e_p = e_ref[...]
    c_p = jax.lax.dot_general(arow_ref[...], e_p, (((1,), (0,)), ((), ())),
                              precision=HIGH, preferred_element_type=jnp.float32)
    b_p = INV_MN / (c_p + scal_ref[0, 0])
    bout_ref[0, 0, :] = b_p[0, :]
    contrib = jax.lax.dot_general(b_p, e_p, (((1,), (1,)), ((), ())),
                                  precision=HIGH, preferred_element_type=jnp.float32)

    @pl.when(t == 0)
    def _():
        racc_ref[...] = contrib

    @pl.when(t != 0)
    def _():
        racc_ref[...] += contrib


def _sweep_last_body(e_ref, arow_ref, acol_ref, scal_ref, bout_ref, w3_ref):
    e_p = e_ref[...]
    c_p = jax.lax.dot_general(arow_ref[...], e_p, (((1,), (0,)), ((), ())),
                              precision=HIGH, preferred_element_type=jnp.float32)
    b_p = INV_MN / (c_p + scal_ref[0, 0])
    bout_ref[0, 0, :] = b_p[0, :]
    w = acol_ref[...] * e_p
    m1 = jnp.max(w, axis=0, keepdims=True)
    w2 = jnp.where(w == m1, -1.0, w)
    m2 = jnp.max(w2, axis=0, keepdims=True)
    w3v = jnp.where(w2 == m2, -1.0, w2)
    m3 = jnp.max(w3v, axis=0, keepdims=True)
    w3_ref[0, 0, :] = m3[0, :]


def _out_body(e_ref, acol_ref, sb_ref, w3_ref, scal_ref, scores_ref, assign_ref):
    t = pl.program_id(0)

    @pl.when(t < 16)
    def _():
        a_t = acol_ref[...]
        w = a_t * e_ref[...]
        s = w * sb_ref[...]
        scores_ref[:, :N] = s
        scores_ref[:, N:N + 1] = scal_ref[0, 2] * a_t
        m1 = jnp.max(s, axis=1, keepdims=True)
        s2 = jnp.where(s == m1, -1.0, s)
        m2 = jnp.max(s2, axis=1, keepdims=True)
        s3 = jnp.where(s2 == m2, -1.0, s2)
        rt3 = jnp.max(s3, axis=1, keepdims=True)
        assign = (s > THRESHOLD) & (s >= rt3) & (w >= w3_ref[...])
        assign_ref[...] = assign

    @pl.when(t == 16)
    def _():
        lastrow = scal_ref[0, 0] * sb_ref[...]
        scores_ref[0:1, :N] = lastrow
        scores_ref[0:1, N:N + 1] = jnp.full((1, 1), 0.0, jnp.float32) + scal_ref[0, 1]


def kernel(desc0, desc1, W, b, alpha):
    f32 = jnp.float32
    mdesc0, mdesc1 = pl.pallas_call(
        _proj_body,
        out_shape=[jax.ShapeDtypeStruct((M, DIM), f32),
                   jax.ShapeDtypeStruct((N, DIM), f32)],
    )(desc0, desc1, W, b.reshape(1, DIM))

    kn, e_mat, s1 = pl.pallas_call(
        _kn_body,
        grid=(M // BM,),
        in_specs=[pl.BlockSpec((BM, DIM), lambda t: (t, 0)),
                  pl.BlockSpec((N, DIM), lambda t: (0, 0))],
        out_specs=[pl.BlockSpec((BM, N), lambda t: (t, 0)),
                   pl.BlockSpec((BM, N), lambda t: (t, 0)),
                   pl.BlockSpec((1, 1, BM), lambda t: (t, 0, 0))],
        out_shape=[jax.ShapeDtypeStruct((M, N), f32),
                   jax.ShapeDtypeStruct((M, N), f32),
                   jax.ShapeDtypeStruct((M // BM, 1, BM), f32)],
    )(mdesc0, mdesc1)

    ea = jnp.exp(alpha)
    s1 = s1.reshape(M)
    a = INV_MN / (s1 + ea)
    a_bin = MU_BIN / (ea * (N + 1.0))

    sweep = pl.pallas_call(
        _sweep_body,
        grid=(NPANEL,),
        in_specs=[pl.BlockSpec((M, PW), lambda t: (0, t)),
                  pl.BlockSpec((1, M), lambda t: (0, 0)),
                  pl.BlockSpec((1, 8), lambda t: (0, 0))],
        out_specs=[pl.BlockSpec((1, 1, PW), lambda t: (t, 0, 0)),
                   pl.BlockSpec((1, N), lambda t: (0, 0))],
        out_shape=[jax.ShapeDtypeStruct((NPANEL, 1, PW), f32),
                   jax.ShapeDtypeStruct((1, N), f32)],
    )

    for _ in range(ITERS - 1):
        scal = jnp.zeros((1, 8), f32) + (ea * a_bin)
        bout, racc = sweep(e_mat, a.reshape(1, M), scal)
        bvec = bout.reshape(N)
        b_bin = MU_BIN / (ea * (jnp.sum(a) + a_bin))
        a_new = INV_MN / (racc.reshape(M) + ea * b_bin)
        a_bin_new = MU_BIN / (ea * (jnp.sum(bvec) + b_bin))
        a, a_bin = a_new, a_bin_new

    scal = jnp.zeros((1, 8), f32) + (ea * a_bin)
    bout, w3 = pl.pallas_call(
        _sweep_last_body,
        grid=(NPANEL,),
        in_specs=[pl.BlockSpec((M, PW), lambda t: (0, t)),
                  pl.BlockSpec((1, M), lambda t: (0, 0)),
                  pl.BlockSpec((M, 1), lambda t: (0, 0)),
                  pl.BlockSpec((1, 8), lambda t: (0, 0))],
        out_specs=[pl.BlockSpec((1, 1, PW), lambda t: (t, 0, 0)),
                   pl.BlockSpec((1, 1, PW), lambda t: (t, 0, 0))],
        out_shape=[jax.ShapeDtypeStruct((NPANEL, 1, PW), f32),
                   jax.ShapeDtypeStruct((NPANEL, 1, PW), f32)],
    )(e_mat, a.reshape(1, M), a.reshape(M, 1), scal)
    bvec = bout.reshape(N)
    b_bin = MU_BIN / (ea * (jnp.sum(a) + a_bin))

    sb = (MN * bvec).reshape(1, N)
    consts = jnp.stack([ea * a_bin,
                        MN * ea * a_bin * b_bin,
                        MN * ea * b_bin,
                        0.0, 0.0, 0.0, 0.0, 0.0]).astype(f32).reshape(1, 8)

    scores, assign = pl.pallas_call(
        _out_body,
        grid=(M // BM + 1,),
        in_specs=[pl.BlockSpec((BM, N), lambda t: (jnp.minimum(t, 15), 0)),
                  pl.BlockSpec((BM, 1), lambda t: (jnp.minimum(t, 15), 0)),
                  pl.BlockSpec((1, N), lambda t: (0, 0)),
                  pl.BlockSpec((1, N), lambda t: (0, 0)),
                  pl.BlockSpec((1, 8), lambda t: (0, 0))],
        out_specs=[pl.BlockSpec((BM, N + 1), lambda t: (t, 0)),
                   pl.BlockSpec((BM, N), lambda t: (jnp.minimum(t, 15), 0))],
        out_shape=[jax.ShapeDtypeStruct((M + 1, N + 1), f32),
                   jax.ShapeDtypeStruct((M, N), jnp.bool_)],
    )(e_mat, a.reshape(M, 1), sb, w3.reshape(1, N), consts)

    return kn, scores, assign


# speedup vs baseline: 27.6023x; 1.8659x over previous
import functools

import jax
import jax.numpy as jnp
from jax.experimental import pallas as pl

M = 4096
N = 4096
DIM = 256
ITERS = 5
THRESHOLD = 0.05
MN = float(M + N)
INV_MN = 1.0 / MN
MU_BIN = N / MN
BM = 256
PW = 512
NPANEL = N // PW
HIGH = jax.lax.Precision.HIGHEST


def _proj_body(d0_ref, d1_ref, w_ref, b_ref, md0_ref, md1_ref):
    w = w_ref[...]
    bias = b_ref[...]
    md0 = jax.lax.dot_general(d0_ref[...], w, (((1,), (1,)), ((), ())),
                              preferred_element_type=jnp.float32)
    md1 = jax.lax.dot_general(d1_ref[...], w, (((1,), (1,)), ((), ())),
                              preferred_element_type=jnp.float32)
    md0_ref[...] = (md0 + bias) * 0.25
    md1_ref[...] = (md1 + bias) * 0.25


def _kn_body(md0_ref, md1_ref, kn_ref, e_ref, s1_ref):
    k = jax.lax.dot_general(md0_ref[...], md1_ref[...], (((1,), (1,)), ((), ())),
                            preferred_element_type=jnp.float32)
    kn_ref[...] = k
    e = jnp.exp(k)
    e_ref[...] = e
    s1_ref[0, 0, :] = jnp.sum(e, axis=1)


def _sweep_body(e_ref, acol_ref, scal_ref, bout_ref, racc_ref):
    t = pl.program_id(0)
    e_p = e_ref[...]
    wc = acol_ref[...] * e_p
    c_p = jnp.sum(wc, axis=0, keepdims=True)
    b_p = INV_MN / (c_p + scal_ref[0, 0])
    bout_ref[0, 0, :] = b_p[0, :]
    contrib = jnp.sum(e_p * b_p, axis=1, keepdims=True)

    @pl.when(t == 0)
    def _():
        racc_ref[...] = contrib

    @pl.when(t != 0)
    def _():
        racc_ref[...] += contrib


def _sweep_last_body(e_ref, acol_ref, scal_ref, bout_ref, w3_ref):
    e_p = e_ref[...]
    w = acol_ref[...] * e_p
    c_p = jnp.sum(w, axis=0, keepdims=True)
    b_p = INV_MN / (c_p + scal_ref[0, 0])
    bout_ref[0, 0, :] = b_p[0, :]
    m1 = jnp.max(w, axis=0, keepdims=True)
    w2 = jnp.where(w == m1, -1.0, w)
    m2 = jnp.max(w2, axis=0, keepdims=True)
    w3v = jnp.where(w2 == m2, -1.0, w2)
    m3 = jnp.max(w3v, axis=0, keepdims=True)
    w3_ref[0, 0, :] = m3[0, :]


def _out_body(e_ref, acol_ref, sb_ref, w3_ref, scal_ref, scores_ref, assign_ref):
    t = pl.program_id(0)

    @pl.when(t < 16)
    def _():
        a_t = acol_ref[...]
        w = a_t * e_ref[...]
        s = w * sb_ref[...]
        scores_ref[:, :N] = s
        scores_ref[:, N:N + 1] = scal_ref[0, 2] * a_t
        m1 = jnp.max(s, axis=1, keepdims=True)
        s2 = jnp.where(s == m1, -1.0, s)
        m2 = jnp.max(s2, axis=1, keepdims=True)
        s3 = jnp.where(s2 == m2, -1.0, s2)
        rt3 = jnp.max(s3, axis=1, keepdims=True)
        assign = (s > THRESHOLD) & (s >= rt3) & (w >= w3_ref[...])
        assign_ref[...] = assign

    @pl.when(t == 16)
    def _():
        lastrow = scal_ref[0, 0] * sb_ref[...]
        scores_ref[0:1, :N] = lastrow
        scores_ref[0:1, N:N + 1] = jnp.full((1, 1), 0.0, jnp.float32) + scal_ref[0, 1]


def kernel(desc0, desc1, W, b, alpha):
    f32 = jnp.float32
    mdesc0, mdesc1 = pl.pallas_call(
        _proj_body,
        out_shape=[jax.ShapeDtypeStruct((M, DIM), f32),
                   jax.ShapeDtypeStruct((N, DIM), f32)],
    )(desc0, desc1, W, b.reshape(1, DIM))

    kn, e_mat, s1 = pl.pallas_call(
        _kn_body,
        grid=(M // BM,),
        in_specs=[pl.BlockSpec((BM, DIM), lambda t: (t, 0)),
                  pl.BlockSpec((N, DIM), lambda t: (0, 0))],
        out_specs=[pl.BlockSpec((BM, N), lambda t: (t, 0)),
                   pl.BlockSpec((BM, N), lambda t: (t, 0)),
                   pl.BlockSpec((1, 1, BM), lambda t: (t, 0, 0))],
        out_shape=[jax.ShapeDtypeStruct((M, N), f32),
                   jax.ShapeDtypeStruct((M, N), f32),
                   jax.ShapeDtypeStruct((M // BM, 1, BM), f32)],
    )(mdesc0, mdesc1)

    ea = jnp.exp(alpha)
    s1 = s1.reshape(M)
    a = INV_MN / (s1 + ea)
    a_bin = MU_BIN / (ea * (N + 1.0))

    sweep = pl.pallas_call(
        _sweep_body,
        grid=(NPANEL,),
        in_specs=[pl.BlockSpec((M, PW), lambda t: (0, t)),
                  pl.BlockSpec((M, 1), lambda t: (0, 0)),
                  pl.BlockSpec((1, 8), lambda t: (0, 0))],
        out_specs=[pl.BlockSpec((1, 1, PW), lambda t: (t, 0, 0)),
                   pl.BlockSpec((M, 1), lambda t: (0, 0))],
        out_shape=[jax.ShapeDtypeStruct((NPANEL, 1, PW), f32),
                   jax.ShapeDtypeStruct((M, 1), f32)],
    )

    for _ in range(ITERS - 1):
        scal = jnp.zeros((1, 8), f32) + (ea * a_bin)
        bout, racc = sweep(e_mat, a.reshape(M, 1), scal)
        bvec = bout.reshape(N)
        b_bin = MU_BIN / (ea * (jnp.sum(a) + a_bin))
        a_new = INV_MN / (racc.reshape(M) + ea * b_bin)
        a_bin_new = MU_BIN / (ea * (jnp.sum(bvec) + b_bin))
        a, a_bin = a_new, a_bin_new

    scal = jnp.zeros((1, 8), f32) + (ea * a_bin)
    bout, w3 = pl.pallas_call(
        _sweep_last_body,
        grid=(NPANEL,),
        in_specs=[pl.BlockSpec((M, PW), lambda t: (0, t)),
                  pl.BlockSpec((M, 1), lambda t: (0, 0)),
                  pl.BlockSpec((1, 8), lambda t: (0, 0))],
        out_specs=[pl.BlockSpec((1, 1, PW), lambda t: (t, 0, 0)),
                   pl.BlockSpec((1, 1, PW), lambda t: (t, 0, 0))],
        out_shape=[jax.ShapeDtypeStruct((NPANEL, 1, PW), f32),
                   jax.ShapeDtypeStruct((NPANEL, 1, PW), f32)],
    )(e_mat, a.reshape(M, 1), scal)
    bvec = bout.reshape(N)
    b_bin = MU_BIN / (ea * (jnp.sum(a) + a_bin))

    sb = (MN * bvec).reshape(1, N)
    consts = jnp.stack([ea * a_bin,
                        MN * ea * a_bin * b_bin,
                        MN * ea * b_bin,
                        0.0, 0.0, 0.0, 0.0, 0.0]).astype(f32).reshape(1, 8)

    scores, assign = pl.pallas_call(
        _out_body,
        grid=(M // BM + 1,),
        in_specs=[pl.BlockSpec((BM, N), lambda t: (jnp.minimum(t, 15), 0)),
                  pl.BlockSpec((BM, 1), lambda t: (jnp.minimum(t, 15), 0)),
                  pl.BlockSpec((1, N), lambda t: (0, 0)),
                  pl.BlockSpec((1, N), lambda t: (0, 0)),
                  pl.BlockSpec((1, 8), lambda t: (0, 0))],
        out_specs=[pl.BlockSpec((BM, N + 1), lambda t: (t, 0)),
                   pl.BlockSpec((BM, N), lambda t: (jnp.minimum(t, 15), 0))],
        out_shape=[jax.ShapeDtypeStruct((M + 1, N + 1), f32),
                   jax.ShapeDtypeStruct((M, N), jnp.bool_)],
    )(e_mat, a.reshape(M, 1), sb, w3.reshape(1, N), consts)

    return kn, scores, assign


# speedup vs baseline: 30.2571x; 1.0962x over previous
import functools

import jax
import jax.numpy as jnp
from jax.experimental import pallas as pl
from jax.experimental.pallas import tpu as pltpu

M = 4096
N = 4096
DIM = 256
ITERS = 5
THRESHOLD = 0.05
MN = float(M + N)
INV_MN = 1.0 / MN
MU_BIN = N / MN
BM = 256
PW = 512
NPANEL = N // PW
HIGH = jax.lax.Precision.HIGHEST


def _proj_body(d0_ref, d1_ref, w_ref, b_ref, md0_ref, md1_ref):
    w = w_ref[...]
    bias = b_ref[...]
    md0 = jax.lax.dot_general(d0_ref[...], w, (((1,), (1,)), ((), ())),
                              preferred_element_type=jnp.float32)
    md1 = jax.lax.dot_general(d1_ref[...], w, (((1,), (1,)), ((), ())),
                              preferred_element_type=jnp.float32)
    md0_ref[...] = (md0 + bias) * 0.25
    md1_ref[...] = (md1 + bias) * 0.25


def _kn_body(md0_ref, md1_ref, kn_ref, e_ref, s1_ref):
    k = jax.lax.dot_general(md0_ref[...], md1_ref[...], (((1,), (1,)), ((), ())),
                            preferred_element_type=jnp.float32)
    kn_ref[...] = k
    e = jnp.exp(k)
    e_ref[...] = e
    s1_ref[0, 0, :] = jnp.sum(e, axis=1)


def _sinkhorn_body(e_ref, a0_ref, scal_ref,
                   bout_ref, w3_ref, aout_ref, sout_ref,
                   a_vm, racc, svec):
    k = pl.program_id(0)
    t = pl.program_id(1)
    ea = scal_ref[0, 0]

    @pl.when(jnp.logical_and(k == 0, t == 0))
    def _():
        a_vm[...] = a0_ref[...]
        svec[0, 0] = scal_ref[0, 1]
        svec[0, 1] = 0.0

    @pl.when(jnp.logical_and(k > 0, t == 0))
    def _():
        a_bin = svec[0, 0]
        b_bin = MU_BIN / (ea * (jnp.sum(a_vm[...]) + a_bin))
        a_vm[...] = INV_MN / (racc[...] + ea * b_bin)
        svec[0, 0] = MU_BIN / (ea * (svec[0, 1] + b_bin))
        svec[0, 1] = 0.0

    @pl.when(jnp.logical_and(k == ITERS - 1, t == 0))
    def _():
        aout_ref[...] = a_vm[...]
        a_bin5 = svec[0, 0]
        b_bin5 = MU_BIN / (ea * (jnp.sum(a_vm[...]) + a_bin5))
        lane = jax.lax.broadcasted_iota(jnp.int32, (1, 8), 1)
        sout_ref[...] = jnp.where(lane == 0, a_bin5, b_bin5)

    e_p = e_ref[...]
    w = a_vm[...] * e_p
    c_p = jnp.sum(w, axis=0, keepdims=True)
    b_p = INV_MN / (c_p + ea * svec[0, 0])
    bout_ref[:, pl.ds(t * PW, PW)] = b_p
    svec[0, 1] += jnp.sum(b_p)

    @pl.when(k < ITERS - 1)
    def _():
        contrib = jnp.sum(e_p * b_p, axis=1, keepdims=True)

        @pl.when(t == 0)
        def _():
            racc[...] = contrib

        @pl.when(t != 0)
        def _():
            racc[...] += contrib

    @pl.when(k == ITERS - 1)
    def _():
        m1 = jnp.max(w, axis=0, keepdims=True)
        w2 = jnp.where(w == m1, -1.0, w)
        m2 = jnp.max(w2, axis=0, keepdims=True)
        w3v = jnp.where(w2 == m2, -1.0, w2)
        m3 = jnp.max(w3v, axis=0, keepdims=True)
        w3_ref[:, pl.ds(t * PW, PW)] = m3


def _out_body(e_ref, acol_ref, sb_ref, w3_ref, scal_ref, scores_ref, assign_ref):
    t = pl.program_id(0)

    @pl.when(t < 16)
    def _():
        a_t = acol_ref[...]
        w = a_t * e_ref[...]
        s = w * sb_ref[...]
        scores_ref[:, :N] = s
        scores_ref[:, N:N + 1] = scal_ref[0, 2] * a_t
        m1 = jnp.max(s, axis=1, keepdims=True)
        s2 = jnp.where(s == m1, -1.0, s)
        m2 = jnp.max(s2, axis=1, keepdims=True)
        s3 = jnp.where(s2 == m2, -1.0, s2)
        rt3 = jnp.max(s3, axis=1, keepdims=True)
        assign = (s > THRESHOLD) & (s >= rt3) & (w >= w3_ref[...])
        assign_ref[...] = assign

    @pl.when(t == 16)
    def _():
        lastrow = scal_ref[0, 0] * sb_ref[...]
        scores_ref[0:1, :N] = lastrow
        scores_ref[0:1, N:N + 1] = jnp.full((1, 1), 0.0, jnp.float32) + scal_ref[0, 1]


def kernel(desc0, desc1, W, b, alpha):
    f32 = jnp.float32
    mdesc0, mdesc1 = pl.pallas_call(
        _proj_body,
        out_shape=[jax.ShapeDtypeStruct((M, DIM), f32),
                   jax.ShapeDtypeStruct((N, DIM), f32)],
    )(desc0, desc1, W, b.reshape(1, DIM))

    kn, e_mat, s1 = pl.pallas_call(
        _kn_body,
        grid=(M // BM,),
        in_specs=[pl.BlockSpec((BM, DIM), lambda t: (t, 0)),
                  pl.BlockSpec((N, DIM), lambda t: (0, 0))],
        out_specs=[pl.BlockSpec((BM, N), lambda t: (t, 0)),
                   pl.BlockSpec((BM, N), lambda t: (t, 0)),
                   pl.BlockSpec((1, 1, BM), lambda t: (t, 0, 0))],
        out_shape=[jax.ShapeDtypeStruct((M, N), f32),
                   jax.ShapeDtypeStruct((M, N), f32),
                   jax.ShapeDtypeStruct((M // BM, 1, BM), f32)],
    )(mdesc0, mdesc1)

    ea = jnp.exp(alpha)
    s1 = s1.reshape(M)
    a = INV_MN / (s1 + ea)
    a_bin = MU_BIN / (ea * (N + 1.0))

    scal_in = jnp.stack([ea, a_bin, 0.0, 0.0, 0.0, 0.0, 0.0, 0.0]).astype(f32).reshape(1, 8)
    bout, w3, a5, sout = pl.pallas_call(
        _sinkhorn_body,
        grid=(ITERS, NPANEL),
        in_specs=[pl.BlockSpec((M, PW), lambda k, t: (0, t)),
                  pl.BlockSpec((M, 1), lambda k, t: (0, 0)),
                  pl.BlockSpec((1, 8), lambda k, t: (0, 0))],
        out_specs=[pl.BlockSpec((1, N), lambda k, t: (0, 0)),
                   pl.BlockSpec((1, N), lambda k, t: (0, 0)),
                   pl.BlockSpec((M, 1), lambda k, t: (0, 0)),
                   pl.BlockSpec((1, 8), lambda k, t: (0, 0))],
        out_shape=[jax.ShapeDtypeStruct((1, N), f32),
                   jax.ShapeDtypeStruct((1, N), f32),
                   jax.ShapeDtypeStruct((M, 1), f32),
                   jax.ShapeDtypeStruct((1, 8), f32)],
        scratch_shapes=[pltpu.VMEM((M, 1), f32),
                        pltpu.VMEM((M, 1), f32),
                        pltpu.SMEM((1, 8), f32)],
    )(e_mat, a.reshape(M, 1), scal_in)

    a_bin5 = sout[0, 0]
    b_bin5 = sout[0, 1]
    sb = MN * bout
    consts = jnp.stack([ea * a_bin5,
                        MN * ea * a_bin5 * b_bin5,
                        MN * ea * b_bin5,
                        0.0, 0.0, 0.0, 0.0, 0.0]).astype(f32).reshape(1, 8)

    scores, assign = pl.pallas_call(
        _out_body,
        grid=(M // BM + 1,),
        in_specs=[pl.BlockSpec((BM, N), lambda t: (jnp.minimum(t, 15), 0)),
                  pl.BlockSpec((BM, 1), lambda t: (jnp.minimum(t, 15), 0)),
                  pl.BlockSpec((1, N), lambda t: (0, 0)),
                  pl.BlockSpec((1, N), lambda t: (0, 0)),
                  pl.BlockSpec((1, 8), lambda t: (0, 0))],
        out_specs=[pl.BlockSpec((BM, N + 1), lambda t: (t, 0)),
                   pl.BlockSpec((BM, N), lambda t: (jnp.minimum(t, 15), 0))],
        out_shape=[jax.ShapeDtypeStruct((M + 1, N + 1), f32),
                   jax.ShapeDtypeStruct((M, N), jnp.bool_)],
    )(e_mat, a5, sb, w3, consts)

    return kn, scores, assign


# speedup vs baseline: 34.8625x; 1.1522x over previous
import jax
import jax.numpy as jnp
from jax.experimental import pallas as pl
from jax.experimental.pallas import tpu as pltpu

M = 4096
N = 4096
DIM = 256
ITERS = 5
THRESHOLD = 0.05
MN = float(M + N)
INV_MN = 1.0 / MN
MU_BIN = N / MN
BM = 256
PW = 512
NPANEL = N // PW
NS1 = 4


def _proj_body(d0_ref, d1_ref, w_ref, b_ref, md0_ref, md1_ref):
    w = w_ref[...]
    bias = b_ref[...]
    md0 = jax.lax.dot_general(d0_ref[...], w, (((1,), (1,)), ((), ())),
                              preferred_element_type=jnp.float32)
    md1 = jax.lax.dot_general(d1_ref[...], w, (((1,), (1,)), ((), ())),
                              preferred_element_type=jnp.float32)
    md0_ref[...] = (md0 + bias) * 0.25
    md1_ref[...] = (md1 + bias) * 0.25


def _kn_body(md0_ref, md1_ref, kn_ref, e16_ref, s1_ref):
    k = jax.lax.dot_general(md0_ref[...], md1_ref[...], (((1,), (1,)), ((), ())),
                            preferred_element_type=jnp.float32)
    kn_ref[...] = k
    e = jnp.exp(k)
    e16_ref[...] = e.astype(jnp.bfloat16)
    s1_ref[...] = jnp.sum(e, axis=1, keepdims=True)


def _sink1_body(e16_ref, s1_ref, scal_ref,
                aout_ref, racc_ref, sc_ref, a_vm, svec):
    k = pl.program_id(0)
    t = pl.program_id(1)
    ea = scal_ref[0, 0]

    @pl.when(jnp.logical_and(k == 0, t == 0))
    def _():
        a_vm[...] = INV_MN / (s1_ref[...] + ea)
        svec[0, 0] = scal_ref[0, 1]
        svec[0, 1] = 0.0

    @pl.when(jnp.logical_and(k > 0, t == 0))
    def _():
        a_bin = svec[0, 0]
        b_bin = MU_BIN / (ea * (jnp.sum(a_vm[...]) + a_bin))
        a_vm[...] = INV_MN / (racc_ref[...] + ea * b_bin)
        svec[0, 0] = MU_BIN / (ea * (svec[0, 1] + b_bin))
        svec[0, 1] = 0.0

    e_p = e16_ref[...]
    a16 = a_vm[...].astype(jnp.bfloat16)
    h = a16 * e_p
    for _ in range(5):
        half = h.shape[0] // 2
        h = h[:half] + h[half:]
    c_p = jnp.sum(h.astype(jnp.float32), axis=0, keepdims=True)
    b_p = INV_MN / (c_p + ea * svec[0, 0])
    svec[0, 1] += jnp.sum(b_p)

    @pl.when(k < NS1 - 1)
    def _():
        g = e16_ref[...] * b_p.astype(jnp.bfloat16)
        g = g[:, :PW // 2] + g[:, PW // 2:]
        g = g[:, :PW // 4] + g[:, PW // 4:]
        contrib = jnp.sum(g.astype(jnp.float32), axis=1, keepdims=True)

        @pl.when(t == 0)
        def _():
            racc_ref[...] = contrib

        @pl.when(t != 0)
        def _():
            racc_ref[...] += contrib

    @pl.when(k == NS1 - 1)
    def _():
        contrib = jnp.sum(e16_ref[...].astype(jnp.float32) * b_p,
                          axis=1, keepdims=True)

        @pl.when(t == 0)
        def _():
            racc_ref[...] = contrib

        @pl.when(t != 0)
        def _():
            racc_ref[...] += contrib

    @pl.when(jnp.logical_and(k == NS1 - 1, t == NPANEL - 1))
    def _():
        aout_ref[...] = a_vm[...]
        lane = jax.lax.broadcasted_iota(jnp.int32, (1, 8), 1)
        sc_ref[...] = jnp.where(lane == 0, svec[0, 0], svec[0, 1])


def _sink2_body(kn_ref, ain_ref, rin_ref, scin_ref, scal_ref,
                bout_ref, w3_ref, aout_ref, sout_ref, a_vm, svec):
    t = pl.program_id(0)
    ea = scal_ref[0, 0]

    @pl.when(t == 0)
    def _():
        a_bin = scin_ref[0, 0]
        b_bin = MU_BIN / (ea * (jnp.sum(ain_ref[...]) + a_bin))
        a_vm[...] = INV_MN / (rin_ref[...] + ea * b_bin)
        a_bin5 = MU_BIN / (ea * (scin_ref[0, 1] + b_bin))
        svec[0, 0] = a_bin5
        aout_ref[...] = a_vm[...]
        b_bin5 = MU_BIN / (ea * (jnp.sum(a_vm[...]) + a_bin5))
        lane = jax.lax.broadcasted_iota(jnp.int32, (1, 8), 1)
        sout_ref[...] = jnp.where(lane == 0, a_bin5, b_bin5)

    e_p = jnp.exp(kn_ref[...])
    w = a_vm[...] * e_p
    c_p = jnp.sum(w, axis=0, keepdims=True)
    b_p = INV_MN / (c_p + ea * svec[0, 0])
    bout_ref[:, pl.ds(t * PW, PW)] = b_p
    m1 = jnp.max(w, axis=0, keepdims=True)
    w2 = jnp.where(w == m1, -1.0, w)
    m2 = jnp.max(w2, axis=0, keepdims=True)
    w3v = jnp.where(w2 == m2, -1.0, w2)
    m3 = jnp.max(w3v, axis=0, keepdims=True)
    w3_ref[:, pl.ds(t * PW, PW)] = m3


def _out_body(kn_ref, acol_ref, b_ref, w3_ref, sout_ref, scal_ref,
              scores_ref, assign_ref):
    t = pl.program_id(0)
    ea = scal_ref[0, 0]
    a_bin5 = sout_ref[0, 0]
    b_bin5 = sout_ref[0, 1]

    @pl.when(t < 16)
    def _():
        a_t = acol_ref[...]
        w = a_t * jnp.exp(kn_ref[...])
        s = w * (MN * b_ref[...])
        scores_ref[:, :N] = s
        scores_ref[:, N:N + 1] = (MN * ea * b_bin5) * a_t
        m1 = jnp.max(s, axis=1, keepdims=True)
        s2 = jnp.where(s == m1, -1.0, s)
        m2 = jnp.max(s2, axis=1, keepdims=True)
        s3 = jnp.where(s2 == m2, -1.0, s2)
        rt3 = jnp.max(s3, axis=1, keepdims=True)
        assign = (s > THRESHOLD) & (s >= rt3) & (w >= w3_ref[...])
        assign_ref[...] = assign

    @pl.when(t == 16)
    def _():
        scores_ref[0:1, :N] = (MN * ea * a_bin5) * b_ref[...]
        scores_ref[0:1, N:N + 1] = jnp.full((1, 1), MN * ea, jnp.float32) * (a_bin5 * b_bin5)


def kernel(desc0, desc1, W, b, alpha):
    f32 = jnp.float32
    mdesc0, mdesc1 = pl.pallas_call(
        _proj_body,
        out_shape=[jax.ShapeDtypeStruct((M, DIM), f32),
                   jax.ShapeDtypeStruct((N, DIM), f32)],
    )(desc0, desc1, W, b.reshape(1, DIM))

    kn, e16, s1 = pl.pallas_call(
        _kn_body,
        grid=(M // BM,),
        in_specs=[pl.BlockSpec((BM, DIM), lambda t: (t, 0)),
                  pl.BlockSpec((N, DIM), lambda t: (0, 0))],
        out_specs=[pl.BlockSpec((BM, N), lambda t: (t, 0)),
                   pl.BlockSpec((BM, N), lambda t: (t, 0)),
                   pl.BlockSpec((BM, 1), lambda t: (t, 0))],
        out_shape=[jax.ShapeDtypeStruct((M, N), f32),
                   jax.ShapeDtypeStruct((M, N), jnp.bfloat16),
                   jax.ShapeDtypeStruct((M, 1), f32)],
    )(mdesc0, mdesc1)

    ea = jnp.exp(alpha)
    scal = jnp.stack([ea, MU_BIN / (ea * (N + 1.0)),
                      0.0, 0.0, 0.0, 0.0, 0.0, 0.0]).astype(f32).reshape(1, 8)

    a3, racc3, sc3 = pl.pallas_call(
        _sink1_body,
        grid=(NS1, NPANEL),
        in_specs=[pl.BlockSpec((M, PW), lambda k, t: (0, t)),
                  pl.BlockSpec((M, 1), lambda k, t: (0, 0)),
                  pl.BlockSpec((1, 8), lambda k, t: (0, 0))],
        out_specs=[pl.BlockSpec((M, 1), lambda k, t: (0, 0)),
                   pl.BlockSpec((M, 1), lambda k, t: (0, 0)),
                   pl.BlockSpec((1, 8), lambda k, t: (0, 0))],
        out_shape=[jax.ShapeDtypeStruct((M, 1), f32),
                   jax.ShapeDtypeStruct((M, 1), f32),
                   jax.ShapeDtypeStruct((1, 8), f32)],
        scratch_shapes=[pltpu.VMEM((M, 1), f32),
                        pltpu.SMEM((1, 8), f32)],
    )(e16, s1, scal)

    bout, w3, a5, sout = pl.pallas_call(
        _sink2_body,
        grid=(NPANEL,),
        in_specs=[pl.BlockSpec((M, PW), lambda t: (0, t)),
                  pl.BlockSpec((M, 1), lambda t: (0, 0)),
                  pl.BlockSpec((M, 1), lambda t: (0, 0)),
                  pl.BlockSpec((1, 8), lambda t: (0, 0)),
                  pl.BlockSpec((1, 8), lambda t: (0, 0))],
        out_specs=[pl.BlockSpec((1, N), lambda t: (0, 0)),
                   pl.BlockSpec((1, N), lambda t: (0, 0)),
                   pl.BlockSpec((M, 1), lambda t: (0, 0)),
                   pl.BlockSpec((1, 8), lambda t: (0, 0))],
        out_shape=[jax.ShapeDtypeStruct((1, N), f32),
                   jax.ShapeDtypeStruct((1, N), f32),
                   jax.ShapeDtypeStruct((M, 1), f32),
                   jax.ShapeDtypeStruct((1, 8), f32)],
        scratch_shapes=[pltpu.VMEM((M, 1), f32),
                        pltpu.SMEM((1, 8), f32)],
    )(kn, a3, racc3, sc3, scal)

    scores, assign = pl.pallas_call(
        _out_body,
        grid=(M // BM + 1,),
        in_specs=[pl.BlockSpec((BM, N), lambda t: (jnp.minimum(t, 15), 0)),
                  pl.BlockSpec((BM, 1), lambda t: (jnp.minimum(t, 15), 0)),
                  pl.BlockSpec((1, N), lambda t: (0, 0)),
                  pl.BlockSpec((1, N), lambda t: (0, 0)),
                  pl.BlockSpec((1, 8), lambda t: (0, 0)),
                  pl.BlockSpec((1, 8), lambda t: (0, 0))],
        out_specs=[pl.BlockSpec((BM, N + 1), lambda t: (t, 0)),
                   pl.BlockSpec((BM, N), lambda t: (jnp.minimum(t, 15), 0))],
        out_shape=[jax.ShapeDtypeStruct((M + 1, N + 1), f32),
                   jax.ShapeDtypeStruct((M, N), jnp.bool_)],
    )(kn, a5, bout, w3, sout, scal)

    return kn, scores, assign


# speedup vs baseline: 36.6888x; 1.0524x over previous
import jax
import jax.numpy as jnp
from jax.experimental import pallas as pl
from jax.experimental.pallas import tpu as pltpu

M = 4096
N = 4096
DIM = 256
ITERS = 5
THRESHOLD = 0.05
MN = float(M + N)
INV_MN = 1.0 / MN
MU_BIN = N / MN
BM = 256
PW = 512
NPANEL = N // PW
PW1 = 1024
NP1 = N // PW1
NS1 = 4


def _proj_body(d0_ref, d1_ref, w_ref, b_ref, md0_ref, md1_ref):
    w = w_ref[...]
    bias = b_ref[...]
    md0 = jax.lax.dot_general(d0_ref[...], w, (((1,), (1,)), ((), ())),
                              preferred_element_type=jnp.float32)
    md1 = jax.lax.dot_general(d1_ref[...], w, (((1,), (1,)), ((), ())),
                              preferred_element_type=jnp.float32)
    md0_ref[...] = (md0 + bias) * 0.25
    md1_ref[...] = (md1 + bias) * 0.25


def _kn_body(md0_ref, md1_ref, kn_ref, e16_ref, s1_ref):
    k = jax.lax.dot_general(md0_ref[...], md1_ref[...], (((1,), (1,)), ((), ())),
                            preferred_element_type=jnp.float32)
    kn_ref[...] = k
    e = jnp.exp(k)
    e16_ref[...] = e.astype(jnp.bfloat16)
    s1_ref[...] = jnp.sum(e, axis=1, keepdims=True)


def _sink1_body(e16_ref, s1_ref, scal_ref,
                aout_ref, racc_ref, sc_ref, a_vm, svec):
    k = pl.program_id(0)
    t = pl.program_id(1)
    ea = scal_ref[0, 0]

    @pl.when(jnp.logical_and(k == 0, t == 0))
    def _():
        a_vm[...] = INV_MN / (s1_ref[...] + ea)
        svec[0, 0] = scal_ref[0, 1]
        svec[0, 1] = 0.0

    @pl.when(jnp.logical_and(k > 0, t == 0))
    def _():
        a_bin = svec[0, 0]
        b_bin = MU_BIN / (ea * (jnp.sum(a_vm[...]) + a_bin))
        a_vm[...] = INV_MN / (racc_ref[...] + ea * b_bin)
        svec[0, 0] = MU_BIN / (ea * (svec[0, 1] + b_bin))
        svec[0, 1] = 0.0

    e_p = e16_ref[...]
    a16 = a_vm[...].astype(jnp.bfloat16)
    h = a16 * e_p
    for _ in range(5):
        half = h.shape[0] // 2
        h = h[:half] + h[half:]
    c_p = jnp.sum(h.astype(jnp.float32), axis=0, keepdims=True)
    b_p = INV_MN / (c_p + ea * svec[0, 0])
    svec[0, 1] += jnp.sum(b_p)

    @pl.when(k < NS1 - 1)
    def _():
        g = e16_ref[...] * b_p.astype(jnp.bfloat16)
        for _ in range(3):
            gh = g.shape[1] // 2
            g = g[:, :gh] + g[:, gh:]
        contrib = jnp.sum(g.astype(jnp.float32), axis=1, keepdims=True)

        @pl.when(t == 0)
        def _():
            racc_ref[...] = contrib

        @pl.when(t != 0)
        def _():
            racc_ref[...] += contrib

    @pl.when(k == NS1 - 1)
    def _():
        contrib = jnp.sum(e16_ref[...].astype(jnp.float32) * b_p,
                          axis=1, keepdims=True)

        @pl.when(t == 0)
        def _():
            racc_ref[...] = contrib

        @pl.when(t != 0)
        def _():
            racc_ref[...] += contrib

    @pl.when(jnp.logical_and(k == NS1 - 1, t == NP1 - 1))
    def _():
        aout_ref[...] = a_vm[...]
        lane = jax.lax.broadcasted_iota(jnp.int32, (1, 8), 1)
        sc_ref[...] = jnp.where(lane == 0, svec[0, 0], svec[0, 1])


def _sink2_body(kn_ref, ain_ref, rin_ref, scin_ref, scal_ref,
                bout_ref, w3_ref, aout_ref, sout_ref, a_vm, svec):
    t = pl.program_id(0)
    ea = scal_ref[0, 0]

    @pl.when(t == 0)
    def _():
        a_bin = scin_ref[0, 0]
        b_bin = MU_BIN / (ea * (jnp.sum(ain_ref[...]) + a_bin))
        a_vm[...] = INV_MN / (rin_ref[...] + ea * b_bin)
        a_bin5 = MU_BIN / (ea * (scin_ref[0, 1] + b_bin))
        svec[0, 0] = a_bin5
        aout_ref[...] = a_vm[...]
        b_bin5 = MU_BIN / (ea * (jnp.sum(a_vm[...]) + a_bin5))
        lane = jax.lax.broadcasted_iota(jnp.int32, (1, 8), 1)
        sout_ref[...] = jnp.where(lane == 0, a_bin5, b_bin5)

    e_p = jnp.exp(kn_ref[...])
    w = a_vm[...] * e_p
    c_p = jnp.sum(w, axis=0, keepdims=True)
    b_p = INV_MN / (c_p + ea * svec[0, 0])
    bout_ref[:, pl.ds(t * PW, PW)] = b_p
    m1 = jnp.max(w, axis=0, keepdims=True)
    w2 = jnp.where(w == m1, -1.0, w)
    m2 = jnp.max(w2, axis=0, keepdims=True)
    w3v = jnp.where(w2 == m2, -1.0, w2)
    m3 = jnp.max(w3v, axis=0, keepdims=True)
    w3_ref[:, pl.ds(t * PW, PW)] = m3


def _out_body(kn_ref, acol_ref, b_ref, w3_ref, sout_ref, scal_ref,
              scores_ref, assign_ref):
    t = pl.program_id(0)
    ea = scal_ref[0, 0]
    a_bin5 = sout_ref[0, 0]
    b_bin5 = sout_ref[0, 1]

    @pl.when(t < 16)
    def _():
        a_t = acol_ref[...]
        w = a_t * jnp.exp(kn_ref[...])
        s = w * (MN * b_ref[...])
        scores_ref[:, :N] = s
        scores_ref[:, N:N + 1] = (MN * ea * b_bin5) * a_t
        m1 = jnp.max(s, axis=1, keepdims=True)
        s2 = jnp.where(s == m1, -1.0, s)
        m2 = jnp.max(s2, axis=1, keepdims=True)
        s3 = jnp.where(s2 == m2, -1.0, s2)
        rt3 = jnp.max(s3, axis=1, keepdims=True)
        assign = (s > THRESHOLD) & (s >= rt3) & (w >= w3_ref[...])
        assign_ref[...] = assign

    @pl.when(t == 16)
    def _():
        scores_ref[0:1, :N] = (MN * ea * a_bin5) * b_ref[...]
        scores_ref[0:1, N:N + 1] = jnp.full((1, 1), MN * ea, jnp.float32) * (a_bin5 * b_bin5)


def kernel(desc0, desc1, W, b, alpha):
    f32 = jnp.float32
    mdesc0, mdesc1 = pl.pallas_call(
        _proj_body,
        out_shape=[jax.ShapeDtypeStruct((M, DIM), f32),
                   jax.ShapeDtypeStruct((N, DIM), f32)],
    )(desc0, desc1, W, b.reshape(1, DIM))

    kn, e16, s1 = pl.pallas_call(
        _kn_body,
        grid=(M // BM,),
        in_specs=[pl.BlockSpec((BM, DIM), lambda t: (t, 0)),
                  pl.BlockSpec((N, DIM), lambda t: (0, 0))],
        out_specs=[pl.BlockSpec((BM, N), lambda t: (t, 0)),
                   pl.BlockSpec((BM, N), lambda t: (t, 0)),
                   pl.BlockSpec((BM, 1), lambda t: (t, 0))],
        out_shape=[jax.ShapeDtypeStruct((M, N), f32),
                   jax.ShapeDtypeStruct((M, N), jnp.bfloat16),
                   jax.ShapeDtypeStruct((M, 1), f32)],
    )(mdesc0, mdesc1)

    ea = jnp.exp(alpha)
    scal = jnp.stack([ea, MU_BIN / (ea * (N + 1.0)),
                      0.0, 0.0, 0.0, 0.0, 0.0, 0.0]).astype(f32).reshape(1, 8)

    a3, racc3, sc3 = pl.pallas_call(
        _sink1_body,
        grid=(NS1, NP1),
        in_specs=[pl.BlockSpec((M, PW1), lambda k, t: (0, t)),
                  pl.BlockSpec((M, 1), lambda k, t: (0, 0)),
                  pl.BlockSpec((1, 8), lambda k, t: (0, 0))],
        out_specs=[pl.BlockSpec((M, 1), lambda k, t: (0, 0)),
                   pl.BlockSpec((M, 1), lambda k, t: (0, 0)),
                   pl.BlockSpec((1, 8), lambda k, t: (0, 0))],
        out_shape=[jax.ShapeDtypeStruct((M, 1), f32),
                   jax.ShapeDtypeStruct((M, 1), f32),
                   jax.ShapeDtypeStruct((1, 8), f32)],
        scratch_shapes=[pltpu.VMEM((M, 1), f32),
                        pltpu.SMEM((1, 8), f32)],
    )(e16, s1, scal)

    bout, w3, a5, sout = pl.pallas_call(
        _sink2_body,
        grid=(NPANEL,),
        in_specs=[pl.BlockSpec((M, PW), lambda t: (0, t)),
                  pl.BlockSpec((M, 1), lambda t: (0, 0)),
                  pl.BlockSpec((M, 1), lambda t: (0, 0)),
                  pl.BlockSpec((1, 8), lambda t: (0, 0)),
                  pl.BlockSpec((1, 8), lambda t: (0, 0))],
        out_specs=[pl.BlockSpec((1, N), lambda t: (0, 0)),
                   pl.BlockSpec((1, N), lambda t: (0, 0)),
                   pl.BlockSpec((M, 1), lambda t: (0, 0)),
                   pl.BlockSpec((1, 8), lambda t: (0, 0))],
        out_shape=[jax.ShapeDtypeStruct((1, N), f32),
                   jax.ShapeDtypeStruct((1, N), f32),
                   jax.ShapeDtypeStruct((M, 1), f32),
                   jax.ShapeDtypeStruct((1, 8), f32)],
        scratch_shapes=[pltpu.VMEM((M, 1), f32),
                        pltpu.SMEM((1, 8), f32)],
    )(kn, a3, racc3, sc3, scal)

    scores, assign = pl.pallas_call(
        _out_body,
        grid=(M // BM + 1,),
        in_specs=[pl.BlockSpec((BM, N), lambda t: (jnp.minimum(t, 15), 0)),
                  pl.BlockSpec((BM, 1), lambda t: (jnp.minimum(t, 15), 0)),
                  pl.BlockSpec((1, N), lambda t: (0, 0)),
                  pl.BlockSpec((1, N), lambda t: (0, 0)),
                  pl.BlockSpec((1, 8), lambda t: (0, 0)),
                  pl.BlockSpec((1, 8), lambda t: (0, 0))],
        out_specs=[pl.BlockSpec((BM, N + 1), lambda t: (t, 0)),
                   pl.BlockSpec((BM, N), lambda t: (jnp.minimum(t, 15), 0))],
        out_shape=[jax.ShapeDtypeStruct((M + 1, N + 1), f32),
                   jax.ShapeDtypeStruct((M, N), jnp.bool_)],
    )(kn, a5, bout, w3, sout, scal)

    return kn, scores, assign


# speedup vs baseline: 36.7408x; 1.0014x over previous
import jax
import jax.numpy as jnp
from jax.experimental import pallas as pl
from jax.experimental.pallas import tpu as pltpu

M = 4096
N = 4096
DIM = 256
ITERS = 5
THRESHOLD = 0.05
MN = float(M + N)
INV_MN = 1.0 / MN
MU_BIN = N / MN
BM = 256
BMO = 256
PW = 512
NPANEL = N // PW
PW1 = 1024
NP1 = N // PW1
NS1 = 4


def _proj_body(d0_ref, d1_ref, w_ref, b_ref, md0_ref, md1_ref):
    w = w_ref[...]
    bias = b_ref[...]
    md0 = jax.lax.dot_general(d0_ref[...], w, (((1,), (1,)), ((), ())),
                              preferred_element_type=jnp.float32)
    md1 = jax.lax.dot_general(d1_ref[...], w, (((1,), (1,)), ((), ())),
                              preferred_element_type=jnp.float32)
    md0_ref[...] = (md0 + bias) * 0.25
    md1_ref[...] = (md1 + bias) * 0.25


def _kn_body(md0_ref, md1_ref, kn_ref, e16_ref, s1_ref):
    k = jax.lax.dot_general(md0_ref[...], md1_ref[...], (((1,), (1,)), ((), ())),
                            preferred_element_type=jnp.float32)
    kn_ref[...] = k
    e = jnp.exp(k)
    e16_ref[...] = e.astype(jnp.bfloat16)
    s1_ref[...] = jnp.sum(e, axis=1, keepdims=True)


def _sink1_body(e16_ref, s1_ref, scal_ref,
                aout_ref, racc_ref, sc_ref, a_vm, svec):
    k = pl.program_id(0)
    t = pl.program_id(1)
    ea = scal_ref[0, 0]

    @pl.when(jnp.logical_and(k == 0, t == 0))
    def _():
        a_vm[...] = INV_MN / (s1_ref[...] + ea)
        svec[0, 0] = scal_ref[0, 1]
        svec[0, 1] = 0.0

    @pl.when(jnp.logical_and(k > 0, t == 0))
    def _():
        a_bin = svec[0, 0]
        b_bin = MU_BIN / (ea * (jnp.sum(a_vm[...]) + a_bin))
        a_vm[...] = INV_MN / (racc_ref[...] + ea * b_bin)
        svec[0, 0] = MU_BIN / (ea * (svec[0, 1] + b_bin))
        svec[0, 1] = 0.0

    e_p = e16_ref[...]
    a16 = a_vm[...].astype(jnp.bfloat16)
    h = a16 * e_p
    for _ in range(5):
        half = h.shape[0] // 2
        h = h[:half] + h[half:]
    c_p = jnp.sum(h.astype(jnp.float32), axis=0, keepdims=True)
    b_p = INV_MN / (c_p + ea * svec[0, 0])
    svec[0, 1] += jnp.sum(b_p)

    @pl.when(k < NS1 - 1)
    def _():
        g = e16_ref[...] * b_p.astype(jnp.bfloat16)
        for _ in range(3):
            gh = g.shape[1] // 2
            g = g[:, :gh] + g[:, gh:]
        contrib = jnp.sum(g.astype(jnp.float32), axis=1, keepdims=True)

        @pl.when(t == 0)
        def _():
            racc_ref[...] = contrib

        @pl.when(t != 0)
        def _():
            racc_ref[...] += contrib

    @pl.when(k == NS1 - 1)
    def _():
        contrib = jnp.sum(e16_ref[...].astype(jnp.float32) * b_p,
                          axis=1, keepdims=True)

        @pl.when(t == 0)
        def _():
            racc_ref[...] = contrib

        @pl.when(t != 0)
        def _():
            racc_ref[...] += contrib

    @pl.when(jnp.logical_and(k == NS1 - 1, t == NP1 - 1))
    def _():
        aout_ref[...] = a_vm[...]
        lane = jax.lax.broadcasted_iota(jnp.int32, (1, 8), 1)
        sc_ref[...] = jnp.where(lane == 0, svec[0, 0], svec[0, 1])


def _sink2_body(kn_ref, ain_ref, rin_ref, scin_ref, scal_ref,
                bout_ref, w3_ref, aout_ref, sout_ref, a_vm, svec):
    t = pl.program_id(0)
    ea = scal_ref[0, 0]

    @pl.when(t == 0)
    def _():
        a_bin = scin_ref[0, 0]
        b_bin = MU_BIN / (ea * (jnp.sum(ain_ref[...]) + a_bin))
        a_vm[...] = INV_MN / (rin_ref[...] + ea * b_bin)
        a_bin5 = MU_BIN / (ea * (scin_ref[0, 1] + b_bin))
        svec[0, 0] = a_bin5
        aout_ref[...] = a_vm[...]
        b_bin5 = MU_BIN / (ea * (jnp.sum(a_vm[...]) + a_bin5))
        lane = jax.lax.broadcasted_iota(jnp.int32, (1, 8), 1)
        sout_ref[...] = jnp.where(lane == 0, a_bin5, b_bin5)

    e_p = jnp.exp(kn_ref[...])
    w = a_vm[...] * e_p
    c_p = jnp.sum(w, axis=0, keepdims=True)
    b_p = INV_MN / (c_p + ea * svec[0, 0])
    bout_ref[:, pl.ds(t * PW, PW)] = b_p
    m1 = jnp.max(w, axis=0, keepdims=True)
    w2 = jnp.where(w == m1, -1.0, w)
    m2 = jnp.max(w2, axis=0, keepdims=True)
    w3v = jnp.where(w2 == m2, -1.0, w2)
    m3 = jnp.max(w3v, axis=0, keepdims=True)
    w3_ref[:, pl.ds(t * PW, PW)] = m3


def _out_body(kn_ref, acol_ref, b_ref, w3_ref, sout_ref, scal_ref,
              scores_ref, assign_ref):
    t = pl.program_id(0)
    ea = scal_ref[0, 0]
    a_bin5 = sout_ref[0, 0]
    b_bin5 = sout_ref[0, 1]

    @pl.when(t < M // BMO)
    def _():
        a_t = acol_ref[...]
        w = a_t * jnp.exp(kn_ref[...])
        s = w * (MN * b_ref[...])
        scores_ref[:, :N] = s
        scores_ref[:, N:N + 1] = (MN * ea * b_bin5) * a_t
        m1 = jnp.max(s, axis=1, keepdims=True)
        s2 = jnp.where(s == m1, -1.0, s)
        m2 = jnp.max(s2, axis=1, keepdims=True)
        s3 = jnp.where(s2 == m2, -1.0, s2)
        rt3 = jnp.max(s3, axis=1, keepdims=True)
        assign = (s > THRESHOLD) & (s >= rt3) & (w >= w3_ref[...])
        assign_ref[...] = assign

    @pl.when(t == M // BMO)
    def _():
        scores_ref[0:1, :N] = (MN * ea * a_bin5) * b_ref[...]
        scores_ref[0:1, N:N + 1] = jnp.full((1, 1), MN * ea, jnp.float32) * (a_bin5 * b_bin5)


def kernel(desc0, desc1, W, b, alpha):
    f32 = jnp.float32
    mdesc0, mdesc1 = pl.pallas_call(
        _proj_body,
        out_shape=[jax.ShapeDtypeStruct((M, DIM), f32),
                   jax.ShapeDtypeStruct((N, DIM), f32)],
    )(desc0, desc1, W, b.reshape(1, DIM))

    kn, e16, s1 = pl.pallas_call(
        _kn_body,
        grid=(M // BM,),
        in_specs=[pl.BlockSpec((BM, DIM), lambda t: (t, 0)),
                  pl.BlockSpec((N, DIM), lambda t: (0, 0))],
        out_specs=[pl.BlockSpec((BM, N), lambda t: (t, 0)),
                   pl.BlockSpec((BM, N), lambda t: (t, 0)),
                   pl.BlockSpec((BM, 1), lambda t: (t, 0))],
        out_shape=[jax.ShapeDtypeStruct((M, N), f32),
                   jax.ShapeDtypeStruct((M, N), jnp.bfloat16),
                   jax.ShapeDtypeStruct((M, 1), f32)],
    )(mdesc0, mdesc1)

    ea = jnp.exp(alpha)
    scal = jnp.stack([ea, MU_BIN / (ea * (N + 1.0)),
                      0.0, 0.0, 0.0, 0.0, 0.0, 0.0]).astype(f32).reshape(1, 8)

    a3, racc3, sc3 = pl.pallas_call(
        _sink1_body,
        grid=(NS1, NP1),
        in_specs=[pl.BlockSpec((M, PW1), lambda k, t: (0, t)),
                  pl.BlockSpec((M, 1), lambda k, t: (0, 0)),
                  pl.BlockSpec((1, 8), lambda k, t: (0, 0))],
        out_specs=[pl.BlockSpec((M, 1), lambda k, t: (0, 0)),
                   pl.BlockSpec((M, 1), lambda k, t: (0, 0)),
                   pl.BlockSpec((1, 8), lambda k, t: (0, 0))],
        out_shape=[jax.ShapeDtypeStruct((M, 1), f32),
                   jax.ShapeDtypeStruct((M, 1), f32),
                   jax.ShapeDtypeStruct((1, 8), f32)],
        scratch_shapes=[pltpu.VMEM((M, 1), f32),
                        pltpu.SMEM((1, 8), f32)],
    )(e16, s1, scal)

    bout, w3, a5, sout = pl.pallas_call(
        _sink2_body,
        grid=(NPANEL,),
        in_specs=[pl.BlockSpec((M, PW), lambda t: (0, t)),
                  pl.BlockSpec((M, 1), lambda t: (0, 0)),
                  pl.BlockSpec((M, 1), lambda t: (0, 0)),
                  pl.BlockSpec((1, 8), lambda t: (0, 0)),
                  pl.BlockSpec((1, 8), lambda t: (0, 0))],
        out_specs=[pl.BlockSpec((1, N), lambda t: (0, 0)),
                   pl.BlockSpec((1, N), lambda t: (0, 0)),
                   pl.BlockSpec((M, 1), lambda t: (0, 0)),
                   pl.BlockSpec((1, 8), lambda t: (0, 0))],
        out_shape=[jax.ShapeDtypeStruct((1, N), f32),
                   jax.ShapeDtypeStruct((1, N), f32),
                   jax.ShapeDtypeStruct((M, 1), f32),
                   jax.ShapeDtypeStruct((1, 8), f32)],
        scratch_shapes=[pltpu.VMEM((M, 1), f32),
                        pltpu.SMEM((1, 8), f32)],
    )(kn, a3, racc3, sc3, scal)

    nbo = M // BMO
    scores, assign = pl.pallas_call(
        _out_body,
        grid=(nbo + 1,),
        in_specs=[pl.BlockSpec((BMO, N), lambda t: (jnp.minimum(t, nbo - 1), 0)),
                  pl.BlockSpec((BMO, 1), lambda t: (jnp.minimum(t, nbo - 1), 0)),
                  pl.BlockSpec((1, N), lambda t: (0, 0)),
                  pl.BlockSpec((1, N), lambda t: (0, 0)),
                  pl.BlockSpec((1, 8), lambda t: (0, 0)),
                  pl.BlockSpec((1, 8), lambda t: (0, 0))],
        out_specs=[pl.BlockSpec((BMO, N + 1), lambda t: (t, 0)),
                   pl.BlockSpec((BMO, N), lambda t: (jnp.minimum(t, nbo - 1), 0))],
        out_shape=[jax.ShapeDtypeStruct((M + 1, N + 1), f32),
                   jax.ShapeDtypeStruct((M, N), jnp.bool_)],
    )(kn, a5, bout, w3, sout, scal)

    return kn, scores, assign
